# baseline (speedup 1.0000x reference)
import functools
import math

import jax
import jax.numpy as jnp
from jax import lax
from jax.experimental import pallas as pl
from jax.experimental.pallas import tpu as pltpu
from jax.experimental.pallas import tpu_sc as plsc

_NC = 2
_NS = 16
_NW = _NC * _NS

_IDXW = 128
_R = 2


@functools.partial(jax.jit, static_argnums=(2,))
def _emb_lookup(table2, idx2d, scale):
    n_rows = idx2d.shape[0]
    rows_per_w = n_rows // _NW
    n_chunks = rows_per_w // _R
    n_pairs = n_chunks // 2
    ch = _R * _IDXW
    out_rows_per_chunk = ch // 2
    out_rows_per_w = rows_per_w * 64

    mesh = plsc.VectorSubcoreMesh(core_axis_name="c", subcore_axis_name="s")

    @functools.partial(
        pl.kernel,
        mesh=mesh,
        out_type=jax.ShapeDtypeStruct((n_rows * 64, 128), jnp.float32),
        scratch_types=[
            pltpu.VMEM((rows_per_w, _IDXW), jnp.int32),
            pltpu.VMEM((2, _R, _IDXW), jnp.int32),
            pltpu.VMEM((2, ch, _IDXW), jnp.float32),
            pltpu.VMEM((out_rows_per_chunk, _IDXW), jnp.float32),
            pltpu.SemaphoreType.DMA,
            pltpu.SemaphoreType.DMA,
        ],
        compiler_params=pltpu.CompilerParams(needs_layout_passes=False),
    )
    def k(table_hbm, idx_hbm, out_hbm, x_all, rowbuf, big, cmp, sem0, sem1):
        wid = lax.axis_index("s") * _NC + lax.axis_index("c")
        sems = (sem0, sem1)
        iota = lax.iota(jnp.int32, 16)
        alt0 = (iota & 1) << 6
        half_iota = lax.shift_right_logical(iota, 1)

        pltpu.sync_copy(idx_hbm.at[pl.ds(wid * rows_per_w, rows_per_w)], x_all)

        def prefetch(g, b):
            for j2 in range(_R):
                xr = g * _R + j2
                for t in range(_IDXW // 16):
                    v = x_all[xr, pl.ds(16 * t, 16)]
                    rowbuf[b, j2, pl.ds(16 * t, 16)] = (
                        lax.shift_right_logical(v, 1))
            for j2 in range(_R):
                pltpu.async_copy(
                    table_hbm.at[rowbuf.at[b].at[j2]],
                    big.at[b].at[pl.ds(j2 * _IDXW, _IDXW)],
                    sems[b],
                )

        def process(g, b):
            for j2 in range(_R):
                pltpu.make_async_copy(
                    table_hbm.at[rowbuf.at[b].at[j2]],
                    big.at[b].at[pl.ds(j2 * _IDXW, _IDXW)],
                    sems[b],
                ).wait()

            def grp_body(grp, c0):
                flat = g * ch + grp * 16
                xv = x_all[lax.shift_right_logical(flat, 7),
                           pl.ds((flat & 127), 16)]
                halfsel = (xv & 1) << 6
                src_row = iota + grp * 16
                dst_row = half_iota + grp * 8

                def j_body(j, cols):
                    c1, c2 = cols
                    v = plsc.load_gather(big.at[b], [src_row, c1])
                    plsc.store_scatter(cmp, [dst_row, c2], v * scale)
                    return (c1 + 1, c2 + 1)

                lax.fori_loop(0, 64, j_body, (halfsel, alt0), unroll=8)
                return c0

            lax.fori_loop(0, ch // 16, grp_body, 0)
            out0 = wid * out_rows_per_w + g * out_rows_per_chunk
            pltpu.sync_copy(cmp, out_hbm.at[pl.ds(out0, out_rows_per_chunk)])

        prefetch(0, 0)

        def pair_body(gp, carry):
            g0 = 2 * gp
            prefetch(g0 + 1, 1)
            process(g0, 0)

            @pl.when(gp + 1 < n_pairs)
            def _():
                prefetch(g0 + 2, 0)

            process(g0 + 1, 1)
            return carry

        lax.fori_loop(0, n_pairs, pair_body, 0)

    return k(table2, idx2d)


def kernel(x, table):
    b = x.shape[0] * x.shape[1]
    d = table.shape[1]
    table2 = table.reshape(table.shape[0] // 2, 2 * d)
    idx2d = x.reshape(b // _IDXW, _IDXW).astype(jnp.int32)
    scale = float(math.sqrt(d))
    out = _emb_lookup(table2, idx2d, scale)
    return out.reshape(x.shape[0], x.shape[1], d)


# speedup vs baseline: 1.8995x; 1.8995x over previous
import functools
import math

import jax
import jax.numpy as jnp
from jax import lax
from jax.experimental import pallas as pl
from jax.experimental.pallas import tpu as pltpu
from jax.experimental.pallas import tpu_sc as plsc

_NC = 2
_NS = 16
_NW = _NC * _NS

_IDXW = 128
_NB = 2


@functools.partial(jax.jit, static_argnums=(2,))
def _emb_lookup(table, xt, scale):
    s_dim, dg = xt.shape
    d = table.shape[1]
    dpw = dg // _NW
    bpw = dpw * s_dim
    nbat = bpw // _IDXW
    ch = _NB * _IDXW
    n_chunks = bpw // ch
    n_pairs = n_chunks // 2
    o_rows = ch * d // 128
    out_rows_w = bpw * d // 128

    mesh = plsc.VectorSubcoreMesh(core_axis_name="c", subcore_axis_name="s")

    @functools.partial(
        pl.kernel,
        mesh=mesh,
        out_type=jax.ShapeDtypeStruct((s_dim * dg * d // 128, 128),
                                      jnp.float32),
        scratch_types=[
            pltpu.VMEM((s_dim, dpw), jnp.int32),
            pltpu.VMEM((nbat, _IDXW), jnp.int32),
            pltpu.VMEM((2, ch, d), jnp.float32),
            pltpu.VMEM((o_rows, 128), jnp.float32),
            pltpu.SemaphoreType.DMA,
            pltpu.SemaphoreType.DMA,
        ],
        compiler_params=pltpu.CompilerParams(
            use_tc_tiling_on_sc=False, needs_layout_passes=False),
    )
    def k(table_hbm, xt_hbm, out_hbm, xs, idxb, gbuf, obuf, sem0, sem1):
        wid = lax.axis_index("s") * _NC + lax.axis_index("c")
        sems = (sem0, sem1)
        iota = lax.iota(jnp.int32, 16)

        pltpu.sync_copy(xt_hbm.at[:, pl.ds(wid * dpw, dpw)], xs)

        for dgrp in range(dpw // 16):
            pbase = (iota + 16 * dgrp) * s_dim

            def t_body(s, p):
                v = xs[s, pl.ds(16 * dgrp, 16)]
                plsc.store_scatter(
                    idxb,
                    [lax.shift_right_logical(p, 7), p & 127],
                    v,
                )
                return p + 1

            lax.fori_loop(0, s_dim, t_body, pbase)

        def fire(g, b):
            for j in range(_NB):
                pltpu.async_copy(
                    table_hbm.at[idxb.at[g * _NB + j]],
                    gbuf.at[b].at[pl.ds(j * _IDXW, _IDXW)],
                    sems[b],
                )

        def process(g, b):
            for j in range(_NB):
                pltpu.make_async_copy(
                    table_hbm.at[idxb.at[j]],
                    gbuf.at[b].at[pl.ds(j * _IDXW, _IDXW)],
                    sems[b],
                ).wait()

            def mul_body(i, c2):
                orow = lax.shift_right_logical(i, 1)
                obase = (i & 1) * d
                for l in range(d // 16):
                    sl = gbuf[b, i, pl.ds(l * 16, 16)]
                    obuf[orow, pl.ds(obase + l * 16, 16)] = sl * scale
                return c2

            lax.fori_loop(0, ch, mul_body, 0)

            out0 = wid * out_rows_w + g * o_rows
            pltpu.sync_copy(obuf, out_hbm.at[pl.ds(out0, o_rows)])

        fire(0, 0)

        def pair_body(gp, carry):
            g0 = 2 * gp
            fire(g0 + 1, 1)
            process(g0, 0)

            @pl.when(gp + 1 < n_pairs)
            def _():
                fire(g0 + 2, 0)

            process(g0 + 1, 1)
            return carry

        lax.fori_loop(0, n_pairs, pair_body, 0)

    return k(table, xt)


def kernel(x, table):
    d = table.shape[1]
    scale = float(math.sqrt(d))
    out = _emb_lookup(table, x.T.astype(jnp.int32), scale)
    return out.reshape(x.shape[0], x.shape[1], d)


# speedup vs baseline: 2.5473x; 1.3410x over previous
import functools
import math

import jax
import jax.numpy as jnp
from jax import lax
from jax.experimental import pallas as pl
from jax.experimental.pallas import tpu as pltpu
from jax.experimental.pallas import tpu_sc as plsc

_NC = 2
_NS = 16
_NW = _NC * _NS

_XR = 2


@functools.partial(jax.jit, static_argnums=(2,))
def _emb_lookup(table, x, scale):
    b0, s_dim = x.shape
    d = table.shape[1]
    rows_w = b0 // _NW
    bpw = rows_w * s_dim
    n_chunks = rows_w // _XR
    n_pairs = n_chunks // 2
    ch = _XR * s_dim
    sub = ((0, 104), (104, 96))
    o_rows = ch * d // 128
    out_rows_w = bpw * d // 128

    mesh = plsc.VectorSubcoreMesh(core_axis_name="c", subcore_axis_name="s")

    @functools.partial(
        pl.kernel,
        mesh=mesh,
        out_type=jax.ShapeDtypeStruct((b0 * s_dim * d // 128, 128),
                                      jnp.float32),
        scratch_types=[
            pltpu.VMEM((rows_w, s_dim), jnp.int32),
            pltpu.VMEM((2, ch, d), jnp.float32),
            pltpu.VMEM((o_rows, 128), jnp.float32),
            pltpu.SemaphoreType.DMA,
            pltpu.SemaphoreType.DMA,
        ],
        compiler_params=pltpu.CompilerParams(
            use_tc_tiling_on_sc=False, needs_layout_passes=False),
    )
    def k(table_hbm, x_hbm, out_hbm, xs, gbuf, obuf, sem0, sem1):
        wid = lax.axis_index("s") * _NC + lax.axis_index("c")
        sems = (sem0, sem1)

        pltpu.sync_copy(x_hbm.at[pl.ds(wid * rows_w, rows_w)], xs)

        def batches(g, b):
            for r in range(_XR):
                for (c0, w) in sub:
                    yield (xs.at[g * _XR + r, pl.ds(c0, w)],
                           gbuf.at[b].at[pl.ds(r * s_dim + c0, w)])

        def fire(g, b):
            for idx_ref, dst in batches(g, b):
                pltpu.async_copy(table_hbm.at[idx_ref], dst, sems[b])

        def process(g, b):
            for idx_ref, dst in batches(g, b):
                pltpu.make_async_copy(table_hbm.at[idx_ref], dst,
                                      sems[b]).wait()

            def mul_body(i2, c2):
                for l in range(d // 16):
                    obuf[i2, pl.ds(l * 16, 16)] = (
                        gbuf[b, 2 * i2, pl.ds(l * 16, 16)] * scale)
                    obuf[i2, pl.ds(d + l * 16, 16)] = (
                        gbuf[b, 2 * i2 + 1, pl.ds(l * 16, 16)] * scale)
                return c2

            lax.fori_loop(0, ch // 2, mul_body, 0)

            out0 = wid * out_rows_w + g * o_rows
            pltpu.sync_copy(obuf, out_hbm.at[pl.ds(out0, o_rows)])

        fire(0, 0)

        def pair_body(gp, carry):
            g0 = 2 * gp
            fire(g0 + 1, 1)
            process(g0, 0)

            @pl.when(gp + 1 < n_pairs)
            def _():
                fire(g0 + 2, 0)

            process(g0 + 1, 1)
            return carry

        lax.fori_loop(0, n_pairs, pair_body, 0)

    return k(table, x)


def kernel(x, table):
    d = table.shape[1]
    scale = float(math.sqrt(d))
    out = _emb_lookup(table, x.astype(jnp.int32), scale)
    return out.reshape(x.shape[0], x.shape[1], d)
